# baseline (speedup 1.0000x reference)
import functools

import jax
import jax.numpy as jnp
from jax import lax
from jax.experimental import pallas as pl
from jax.experimental.pallas import tpu as pltpu
from jax.experimental.pallas import tpu_sc as plsc

_NC = 2
_NS = 16
_L = 16
_NW = _NC * _NS
_SUB = 4
_UNROLL = 8
_BLK = 32768


@functools.lru_cache(maxsize=None)
def _make_hist_kernel(sc_n: int, sc_base: int):
    chunk = sc_n // _NW
    sub = chunk // _SUB
    mesh = plsc.VectorSubcoreMesh(
        core_axis_name="c", subcore_axis_name="s", num_cores=_NC,
        num_subcores=_NS)

    @functools.partial(
        pl.kernel,
        out_type=jax.ShapeDtypeStruct((2 * _NW * _L,), jnp.float32),
        mesh=mesh,
        compiler_params=pltpu.CompilerParams(needs_layout_passes=False),
        scratch_types=[
            pltpu.VMEM((chunk,), jnp.int32),
            pltpu.VMEM((chunk,), jnp.int32),
            pltpu.VMEM((2 * _L * _L,), jnp.float32),
            pltpu.VMEM((_L,), jnp.float32),
            pltpu.SemaphoreType.DMA((_SUB,)),
        ],
    )
    def hist_kernel(preds_hbm, attrs_hbm, out_hbm, preds_v, attrs_v, hist_v,
                    row_v, sems):
        c = lax.axis_index("c")
        s = lax.axis_index("s")
        wid = s * _NC + c
        base = sc_base + wid * chunk

        descs = []
        for g in range(_SUB):
            src_p = preds_hbm.at[pl.ds(base + g * sub, sub)]
            src_a = attrs_hbm.at[pl.ds(base + g * sub, sub)]
            descs.append(
                (pltpu.async_copy(src_p, preds_v.at[pl.ds(g * sub, sub)],
                                  sems.at[g]),
                 pltpu.async_copy(src_a, attrs_v.at[pl.ds(g * sub, sub)],
                                  sems.at[g])))

        zeros = jnp.zeros((_L,), jnp.float32)
        for i in range(2 * _L):
            hist_v[pl.ds(i * _L, _L)] = zeros

        lane16 = lax.iota(jnp.int32, _L) * _L
        ones = jnp.ones((_L,), jnp.float32)

        for g in range(_SUB):
            descs[g][0].wait()
            descs[g][1].wait()

            @functools.partial(
                plsc.parallel_loop, 0, sub // _L, unroll=_UNROLL)
            def body(i, g=g):
                off = g * sub + i * _L
                p = preds_v[pl.ds(off, _L)]
                a = attrs_v[pl.ds(off, _L)]
                addr = a * 256 + (lane16 + p)
                plsc.addupdate_scatter(hist_v, [addr], ones)

        for a in range(2):
            acc = hist_v[pl.ds(a * 256, _L)]
            for l in range(1, _L):
                acc = acc + hist_v[pl.ds(a * 256 + l * _L, _L)]
            row_v[...] = acc
            pltpu.sync_copy(row_v,
                            out_hbm.at[pl.ds((a * _NW + wid) * _L, _L)])

    return hist_kernel


def _tc_hist_body(p_ref, a_ref, o_ref):
    i = pl.program_id(0)
    first = i == 0
    pv = p_ref[...].reshape(_BLK // 128, 128)
    av = a_ref[...].reshape(_BLK // 128, 128).astype(jnp.float32)
    one = jnp.ones_like(av)
    zero = jnp.zeros_like(av)
    for p in range(9):
        m = pv == p
        tot = jnp.sum(jnp.where(m, one, zero), axis=0)
        c1 = jnp.sum(jnp.where(m, av, zero), axis=0)
        o_ref[p, :] = jnp.where(first, tot, o_ref[p, :] + tot)
        o_ref[_L + p, :] = jnp.where(first, c1, o_ref[_L + p, :] + c1)


def _make_tc_hist(tc_n: int, off_blocks: int):
    return pl.pallas_call(
        _tc_hist_body,
        grid=(tc_n // _BLK,),
        in_specs=[
            pl.BlockSpec((_BLK,), lambda i: (off_blocks + i,)),
            pl.BlockSpec((_BLK,), lambda i: (off_blocks + i,)),
        ],
        out_specs=pl.BlockSpec((2 * _L, 128), lambda i: (0, 0)),
        out_shape=jax.ShapeDtypeStruct((2 * _L, 128), jnp.float32),
    )


def _finalize_body(x_ref, t_ref, o_ref):
    h0 = x_ref[pl.ds(0, _L)]
    h1 = x_ref[pl.ds(_NW * _L, _L)]
    for w in range(1, _NW):
        h0 = h0 + x_ref[pl.ds(w * _L, _L)]
        h1 = h1 + x_ref[pl.ds((_NW + w) * _L, _L)]
    lane = lax.broadcasted_iota(jnp.int32, (_L,), 0)
    for p in range(9):
        tot = jnp.sum(t_ref[p, :])
        c1 = jnp.sum(t_ref[_L + p, :])
        h0 = jnp.where(lane == p, h0 + (tot - c1), h0)
        h1 = jnp.where(lane == p, h1 + c1, h1)
    n0 = jnp.sum(h0)
    n1 = jnp.sum(h1)
    d = h0 / n0 - h1 / n1
    o_ref[0, 0] = jnp.sum(d * d)


@jax.jit
def kernel(preds, attrs):
    n = preds.shape[0]
    tc_n = n // 2
    sc_n = n - tc_n
    partial = _make_hist_kernel(sc_n, tc_n)(preds, attrs)
    tc_counts = _make_tc_hist(tc_n, 0)(preds, attrs)
    spd = pl.pallas_call(
        _finalize_body,
        in_specs=[
            pl.BlockSpec(memory_space=pltpu.VMEM),
            pl.BlockSpec(memory_space=pltpu.VMEM),
        ],
        out_shape=jax.ShapeDtypeStruct((1, 1), jnp.float32),
        out_specs=pl.BlockSpec(memory_space=pltpu.SMEM),
    )(partial, tc_counts)
    return spd[0, 0]


# speedup vs baseline: 1.1568x; 1.1568x over previous
import functools

import jax
import jax.numpy as jnp
from jax import lax
from jax.experimental import pallas as pl
from jax.experimental.pallas import tpu as pltpu
from jax.experimental.pallas import tpu_sc as plsc

_NC = 2
_NS = 16
_L = 16
_NW = _NC * _NS
_SUB = 4
_UNROLL = 8


@functools.lru_cache(maxsize=None)
def _make_hist_kernel(n: int):
    chunk = n // _NW
    sub = chunk // _SUB
    mesh = plsc.VectorSubcoreMesh(
        core_axis_name="c", subcore_axis_name="s", num_cores=_NC,
        num_subcores=_NS)

    @functools.partial(
        pl.kernel,
        out_type=jax.ShapeDtypeStruct((2 * _NW * _L,), jnp.float32),
        mesh=mesh,
        compiler_params=pltpu.CompilerParams(needs_layout_passes=False),
        scratch_types=[
            pltpu.VMEM((chunk,), jnp.int32),
            pltpu.VMEM((chunk,), jnp.int32),
            pltpu.VMEM((2 * _L * _L,), jnp.float32),
            pltpu.VMEM((_L,), jnp.float32),
            pltpu.SemaphoreType.DMA((_SUB,)),
        ],
    )
    def hist_kernel(preds_hbm, attrs_hbm, out_hbm, preds_v, attrs_v, hist_v,
                    row_v, sems):
        c = lax.axis_index("c")
        s = lax.axis_index("s")
        wid = s * _NC + c
        base = wid * chunk

        descs = []
        for g in range(_SUB):
            src_p = preds_hbm.at[pl.ds(base + g * sub, sub)]
            src_a = attrs_hbm.at[pl.ds(base + g * sub, sub)]
            descs.append(
                (pltpu.async_copy(src_p, preds_v.at[pl.ds(g * sub, sub)],
                                  sems.at[g]),
                 pltpu.async_copy(src_a, attrs_v.at[pl.ds(g * sub, sub)],
                                  sems.at[g])))

        zeros = jnp.zeros((_L,), jnp.float32)
        for i in range(2 * _L):
            hist_v[pl.ds(i * _L, _L)] = zeros

        lane16 = lax.iota(jnp.int32, _L) * _L
        ones = jnp.ones((_L,), jnp.float32)

        for g in range(_SUB):
            descs[g][0].wait()
            descs[g][1].wait()

            @functools.partial(
                plsc.parallel_loop, 0, sub // _L, unroll=_UNROLL)
            def body(i, g=g):
                off = g * sub + i * _L
                p = preds_v[pl.ds(off, _L)]
                a = attrs_v[pl.ds(off, _L)]
                addr = a * 256 + (lane16 + p)
                plsc.addupdate_scatter(hist_v, [addr], ones)

        for a in range(2):
            acc = hist_v[pl.ds(a * 256, _L)]
            for l in range(1, _L):
                acc = acc + hist_v[pl.ds(a * 256 + l * _L, _L)]
            row_v[...] = acc
            pltpu.sync_copy(row_v,
                            out_hbm.at[pl.ds((a * _NW + wid) * _L, _L)])

    return hist_kernel


def _spd_body(x_ref, o_ref):
    h0 = x_ref[pl.ds(0, _L)]
    h1 = x_ref[pl.ds(_NW * _L, _L)]
    for w in range(1, _NW):
        h0 = h0 + x_ref[pl.ds(w * _L, _L)]
        h1 = h1 + x_ref[pl.ds((_NW + w) * _L, _L)]
    n0 = jnp.sum(h0)
    n1 = jnp.sum(h1)
    d = h0 / n0 - h1 / n1
    o_ref[0, 0] = jnp.sum(d * d)


@jax.jit
def kernel(preds, attrs):
    n = preds.shape[0]
    partial = _make_hist_kernel(n)(preds, attrs)
    spd = pl.pallas_call(
        _spd_body,
        out_shape=jax.ShapeDtypeStruct((1, 1), jnp.float32),
        out_specs=pl.BlockSpec(memory_space=pltpu.SMEM),
    )(partial)
    return spd[0, 0]
